# baseline (speedup 1.0000x reference)
import jax
import jax.numpy as jnp
from jax.experimental import pallas as pl
from jax.experimental.pallas import tpu as pltpu

_CHUNK = 512
_RING = 24


def _make_body(total_rows, rows_per_batch):
    n = total_rows // _CHUNK
    anchor_every = rows_per_batch // _CHUNK

    def _body(emb_ref, anchor_ref, out_ref, bufs, anchor_v,
              load_sems, store_sems, anchor_sem):
        acp = pltpu.make_async_copy(anchor_ref, anchor_v, anchor_sem)
        acp.start()
        loads = {}
        stores = {}
        for i in range(min(_RING, n)):
            loads[i] = pltpu.make_async_copy(
                emb_ref.at[pl.ds(i * _CHUNK, _CHUNK), :],
                bufs.at[i % _RING],
                load_sems.at[i % _RING],
            )
            loads[i].start()
        acp.wait()
        for i in range(n):
            loads[i].wait()
            if i % anchor_every == 0:
                bufs[i % _RING, 0:1, :] = anchor_v[...]
            stores[i] = pltpu.make_async_copy(
                bufs.at[i % _RING],
                out_ref.at[pl.ds(i * _CHUNK, _CHUNK), :],
                store_sems.at[i % _RING],
            )
            stores[i].start()
            nxt = i + _RING
            if nxt < n:
                stores[i].wait()
                loads[nxt] = pltpu.make_async_copy(
                    emb_ref.at[pl.ds(nxt * _CHUNK, _CHUNK), :],
                    bufs.at[nxt % _RING],
                    load_sems.at[nxt % _RING],
                )
                loads[nxt].start()
        for i in range(max(0, n - _RING), n):
            stores[i].wait()

    return _body


def kernel(token_embeddings, style_anchor):
    B, S, D = token_embeddings.shape
    flat = token_embeddings.reshape(B * S, D)
    out = pl.pallas_call(
        _make_body(B * S, S),
        in_specs=[
            pl.BlockSpec(memory_space=pl.ANY),
            pl.BlockSpec(memory_space=pl.ANY),
        ],
        out_specs=pl.BlockSpec(memory_space=pl.ANY),
        out_shape=jax.ShapeDtypeStruct(flat.shape, flat.dtype),
        scratch_shapes=[
            pltpu.VMEM((_RING, _CHUNK, D), flat.dtype),
            pltpu.VMEM((1, D), flat.dtype),
            pltpu.SemaphoreType.DMA((_RING,)),
            pltpu.SemaphoreType.DMA((_RING,)),
            pltpu.SemaphoreType.DMA,
        ],
    )(flat, style_anchor)
    return out.reshape(B, S, D)


# speedup vs baseline: 1.0450x; 1.0450x over previous
import jax
import jax.numpy as jnp
from jax.experimental import pallas as pl
from jax.experimental.pallas import tpu as pltpu

_CHUNK = 1024
_RING = 8


def _make_body(total_rows, rows_per_batch):
    n = total_rows // _CHUNK
    anchor_every = rows_per_batch // _CHUNK

    def _body(emb_ref, anchor_ref, out_ref, bufs, anchor_v,
              load_sems, store_sems, anchor_sem):
        acp = pltpu.make_async_copy(anchor_ref, anchor_v, anchor_sem)
        acp.start()
        loads = {}
        stores = {}
        for i in range(min(_RING, n)):
            loads[i] = pltpu.make_async_copy(
                emb_ref.at[pl.ds(i * _CHUNK, _CHUNK), :],
                bufs.at[i % _RING],
                load_sems.at[i % _RING],
            )
            loads[i].start()
        acp.wait()
        for i in range(n):
            loads[i].wait()
            if i % anchor_every == 0:
                bufs[i % _RING, 0:1, :] = anchor_v[...]
            stores[i] = pltpu.make_async_copy(
                bufs.at[i % _RING],
                out_ref.at[pl.ds(i * _CHUNK, _CHUNK), :],
                store_sems.at[i % _RING],
            )
            stores[i].start()
            nxt = i + _RING
            if nxt < n:
                stores[i].wait()
                loads[nxt] = pltpu.make_async_copy(
                    emb_ref.at[pl.ds(nxt * _CHUNK, _CHUNK), :],
                    bufs.at[nxt % _RING],
                    load_sems.at[nxt % _RING],
                )
                loads[nxt].start()
        for i in range(max(0, n - _RING), n):
            stores[i].wait()

    return _body


def kernel(token_embeddings, style_anchor):
    B, S, D = token_embeddings.shape
    flat = token_embeddings.reshape(B * S, D)
    out = pl.pallas_call(
        _make_body(B * S, S),
        in_specs=[
            pl.BlockSpec(memory_space=pl.ANY),
            pl.BlockSpec(memory_space=pl.ANY),
        ],
        out_specs=pl.BlockSpec(memory_space=pl.ANY),
        out_shape=jax.ShapeDtypeStruct(flat.shape, flat.dtype),
        scratch_shapes=[
            pltpu.VMEM((_RING, _CHUNK, D), flat.dtype),
            pltpu.VMEM((1, D), flat.dtype),
            pltpu.SemaphoreType.DMA((_RING,)),
            pltpu.SemaphoreType.DMA((_RING,)),
            pltpu.SemaphoreType.DMA,
        ],
    )(flat, style_anchor)
    return out.reshape(B, S, D)


# speedup vs baseline: 1.0790x; 1.0325x over previous
import jax
import jax.numpy as jnp
from jax.experimental import pallas as pl
from jax.experimental.pallas import tpu as pltpu

_CHUNK = 1024
_RING = 14


def _make_body(total_rows, rows_per_batch):
    n = total_rows // _CHUNK
    anchor_every = rows_per_batch // _CHUNK

    def _body(emb_ref, anchor_ref, out_ref, bufs, anchor_v,
              load_sems, store_sems, anchor_sem):
        acp = pltpu.make_async_copy(anchor_ref, anchor_v, anchor_sem)
        acp.start()
        loads = {}
        stores = {}
        for i in range(min(_RING, n)):
            loads[i] = pltpu.make_async_copy(
                emb_ref.at[pl.ds(i * _CHUNK, _CHUNK), :],
                bufs.at[i % _RING],
                load_sems.at[i % _RING],
            )
            loads[i].start()
        acp.wait()
        for i in range(n):
            loads[i].wait()
            if i % anchor_every == 0:
                bufs[i % _RING, 0:1, :] = anchor_v[...]
            stores[i] = pltpu.make_async_copy(
                bufs.at[i % _RING],
                out_ref.at[pl.ds(i * _CHUNK, _CHUNK), :],
                store_sems.at[i % _RING],
            )
            stores[i].start()
            nxt = i + _RING
            if nxt < n:
                stores[i].wait()
                loads[nxt] = pltpu.make_async_copy(
                    emb_ref.at[pl.ds(nxt * _CHUNK, _CHUNK), :],
                    bufs.at[nxt % _RING],
                    load_sems.at[nxt % _RING],
                )
                loads[nxt].start()
        for i in range(max(0, n - _RING), n):
            stores[i].wait()

    return _body


def kernel(token_embeddings, style_anchor):
    B, S, D = token_embeddings.shape
    flat = token_embeddings.reshape(B * S, D)
    out = pl.pallas_call(
        _make_body(B * S, S),
        in_specs=[
            pl.BlockSpec(memory_space=pl.ANY),
            pl.BlockSpec(memory_space=pl.ANY),
        ],
        out_specs=pl.BlockSpec(memory_space=pl.ANY),
        out_shape=jax.ShapeDtypeStruct(flat.shape, flat.dtype),
        scratch_shapes=[
            pltpu.VMEM((_RING, _CHUNK, D), flat.dtype),
            pltpu.VMEM((1, D), flat.dtype),
            pltpu.SemaphoreType.DMA((_RING,)),
            pltpu.SemaphoreType.DMA((_RING,)),
            pltpu.SemaphoreType.DMA,
        ],
    )(flat, style_anchor)
    return out.reshape(B, S, D)
